# baseline (speedup 1.0000x reference)
import functools

import jax
import jax.numpy as jnp
from jax import lax
from jax.experimental import pallas as pl
from jax.experimental.pallas import tpu as pltpu
from jax.experimental.pallas import tpu_sc as plsc

EPS = 0.1
N = 10000
D = 128
E = 320000
NC = 2
NS = 16
NW = NC * NS
EPT = E // NW
CH = 125
NCHUNK = EPT // CH
NGRP = NCHUNK // 8
RPT = 624
BLK = 1000
GRID = N // BLK

_mesh = plsc.VectorSubcoreMesh(core_axis_name="c", subcore_axis_name="s")


def _hop_body(s_hbm, row3, col3, outa, outb,
              rv, colv, rows0, rows1, zbuf, acc, semz, sem0, sem1, semr):
    c = lax.axis_index("c")
    sid = lax.axis_index("s")
    wid = c * NS + sid

    for r in range(16):
        for k in range(D // 16):
            zbuf[r, pl.ds(k * 16, 16)] = jnp.zeros((16,), jnp.float32)

    zdescs = [pltpu.async_copy(zbuf, acc.at[pl.ds(sid * RPT + j * 16, 16)],
                               semz) for j in range(RPT // 16)]

    pltpu.sync_copy(col3.at[wid], colv)
    pltpu.sync_copy(row3.at[wid, pl.ds(0, 8)], rv.at[pl.ds(0, 8)])

    @pl.when(sid == NS - 1)
    def _():
        pltpu.async_copy(zbuf, acc.at[pl.ds(N - 16, 16)], semz).wait()
    for d in zdescs:
        d.wait()
    plsc.subcore_barrier()

    def _gather(slot, buf, sem):
        return pltpu.async_copy(s_hbm.at[rv.at[slot]], buf, sem)

    def _scatter(i, buf):
        pltpu.sync_copy(buf, acc.at[colv.at[i]], add=True)

    def _group(g, _):
        p = (g % 2) * 8
        gnext = pl.multiple_of(jnp.minimum(g + 1, NGRP - 1) * 8, 8)
        dpre = pltpu.async_copy(row3.at[wid, pl.ds(gnext, 8)],
                                rv.at[pl.ds(8 - p, 8)], semr)

        d0 = _gather(p, rows0, sem0)
        for k in range(4):
            i0 = g * 8 + 2 * k
            d1 = _gather(p + 2 * k + 1, rows1, sem1)
            d0.wait()
            _scatter(i0, rows0)
            if k < 3:
                d0 = _gather(p + 2 * k + 2, rows0, sem0)
            d1.wait()
            _scatter(i0 + 1, rows1)

        dpre.wait()
        return 0

    lax.fori_loop(0, NGRP, _group, 0)
    plsc.subcore_barrier()

    def _writeback(out):
        pltpu.sync_copy(acc.at[pl.ds(sid * RPT, RPT)],
                        out.at[pl.ds(sid * RPT, RPT)])

        @pl.when(sid == NS - 1)
        def _():
            pltpu.sync_copy(acc.at[pl.ds(N - 16, 16)],
                            out.at[pl.ds(N - 16, 16)])

    @pl.when(c == 0)
    def _():
        _writeback(outa)

    @pl.when(c == 1)
    def _():
        _writeback(outb)


_sc_hop_raw = functools.partial(
    pl.kernel,
    out_type=[jax.ShapeDtypeStruct((N, D), jnp.float32),
              jax.ShapeDtypeStruct((N, D), jnp.float32)],
    mesh=_mesh,
    scratch_types=[
        pltpu.VMEM((16, CH), jnp.int32),
        pltpu.VMEM((NCHUNK, CH), jnp.int32),
        pltpu.VMEM((CH, D), jnp.float32),
        pltpu.VMEM((CH, D), jnp.float32),
        pltpu.VMEM((16, D), jnp.float32),
        pltpu.VMEM_SHARED((N, D), jnp.float32),
        pltpu.SemaphoreType.DMA,
        pltpu.SemaphoreType.DMA,
        pltpu.SemaphoreType.DMA,
        pltpu.SemaphoreType.DMA,
    ],
)(_hop_body)


def _sc_hop(s, row3, col3):
    return _sc_hop_raw(s, row3, col3)


def _scale_body(x_ref, d_ref, o_ref):
    o_ref[...] = x_ref[...] * d_ref[...]


_k_scale = pl.pallas_call(
    _scale_body,
    grid=(GRID,),
    in_specs=[pl.BlockSpec((BLK, D), lambda i: (i, 0)),
              pl.BlockSpec((BLK, 1), lambda i: (i, 0))],
    out_specs=pl.BlockSpec((BLK, D), lambda i: (i, 0)),
    out_shape=jax.ShapeDtypeStruct((N, D), jnp.float32),
)


def _mid_body(ta_ref, tb_ref, d2_ref, o_ref):
    o_ref[...] = d2_ref[...] * (ta_ref[...] + tb_ref[...])


_k_mid = pl.pallas_call(
    _mid_body,
    grid=(GRID,),
    in_specs=[pl.BlockSpec((BLK, D), lambda i: (i, 0)),
              pl.BlockSpec((BLK, D), lambda i: (i, 0)),
              pl.BlockSpec((BLK, 1), lambda i: (i, 0))],
    out_specs=pl.BlockSpec((BLK, D), lambda i: (i, 0)),
    out_shape=jax.ShapeDtypeStruct((N, D), jnp.float32),
)


def _make_step(cfac):
    def _step_body(hs_ref, hb_ref, t1a, t1b, t2a, t2b, d_ref,
                   w0, w1, w2, b_ref, ho_ref, so_ref):
        dv = d_ref[...]
        cur1 = dv * (t1a[...] + t1b[...])
        cur2 = dv * (t2a[...] + t2b[...])
        conv = jnp.dot(hs_ref[...], w0[...], preferred_element_type=jnp.float32)
        conv = conv + jnp.dot(cur1, w1[...], preferred_element_type=jnp.float32)
        conv = conv + jnp.dot(cur2, w2[...], preferred_element_type=jnp.float32)
        conv = conv + b_ref[...]
        ho = hb_ref[...] + cfac * jnp.tanh(conv)
        ho_ref[...] = ho
        so_ref[...] = dv * ho

    blk = pl.BlockSpec((BLK, D), lambda i: (i, 0))
    return pl.pallas_call(
        _step_body,
        grid=(GRID,),
        in_specs=[blk, blk, blk, blk, blk, blk,
                  pl.BlockSpec((BLK, 1), lambda i: (i, 0)),
                  pl.BlockSpec((D, D), lambda i: (0, 0)),
                  pl.BlockSpec((D, D), lambda i: (0, 0)),
                  pl.BlockSpec((D, D), lambda i: (0, 0)),
                  pl.BlockSpec((1, D), lambda i: (0, 0))],
        out_specs=[blk, blk],
        out_shape=[jax.ShapeDtypeStruct((N, D), jnp.float32),
                   jax.ShapeDtypeStruct((N, D), jnp.float32)],
    )


_k_step_mid = _make_step(0.5 * EPS)
_k_step_full = _make_step(EPS)


def _readout_body(hm_ref, wr_ref, br_ref, y_ref):
    y_ref[...] = (jnp.dot(hm_ref[...], wr_ref[...],
                          preferred_element_type=jnp.float32) + br_ref[...])


_k_readout = pl.pallas_call(
    _readout_body,
    grid=(GRID,),
    in_specs=[pl.BlockSpec((BLK, D), lambda i: (i, 0)),
              pl.BlockSpec((D, D), lambda i: (0, 0)),
              pl.BlockSpec((1, D), lambda i: (0, 0))],
    out_specs=pl.BlockSpec((BLK, D), lambda i: (i, 0)),
    out_shape=jax.ShapeDtypeStruct((N, D), jnp.float32),
)


def kernel(x, edge_index, delta_t, W0, W1, W2, b, Wr, br):
    row3 = edge_index[0].reshape(NW, NCHUNK, CH)
    col3 = edge_index[1].reshape(NW, NCHUNK, CH)

    dega, degb = _sc_hop(jnp.ones((N, D), jnp.float32), row3, col3)
    deg = dega[:, 0] + degb[:, 0]
    dinv = jnp.where(deg > 0, lax.rsqrt(jnp.where(deg > 0, deg, 1.0)), 0.0)
    dcol = dinv.reshape(N, 1)
    d2col = dcol * dcol
    b2 = b.reshape(1, D)
    br2 = br.reshape(1, D)

    s0 = _k_scale(x, dcol)

    def _step(_, carry):
        h, hm, s = carry
        t1a, t1b = _sc_hop(s, row3, col3)
        s1 = _k_mid(t1a, t1b, d2col)
        t2a, t2b = _sc_hop(s1, row3, col3)
        hm_new, sm = _k_step_mid(h, h, t1a, t1b, t2a, t2b, dcol,
                                 W0, W1, W2, b2)
        t3a, t3b = _sc_hop(sm, row3, col3)
        s3 = _k_mid(t3a, t3b, d2col)
        t4a, t4b = _sc_hop(s3, row3, col3)
        h_new, s_new = _k_step_full(hm_new, h, t3a, t3b, t4a, t4b, dcol,
                                    W0, W1, W2, b2)
        return (h_new, hm_new, s_new)

    h, hm, _ = lax.fori_loop(0, delta_t, _step, (x, x, s0))
    y = _k_readout(hm, Wr, br2)
    return (y, hm)


# speedup vs baseline: 1.0222x; 1.0222x over previous
import functools

import jax
import jax.numpy as jnp
from jax import lax
from jax.experimental import pallas as pl
from jax.experimental.pallas import tpu as pltpu
from jax.experimental.pallas import tpu_sc as plsc

EPS = 0.1
N = 10000
D = 128
E = 320000
NC = 2
NS = 16
NW = NC * NS
EPT = E // NW
CH = 125
NCHUNK = EPT // CH
NGRP = NCHUNK // 8
RPT = 624
BLK = 1000
GRID = N // BLK

_mesh = plsc.VectorSubcoreMesh(core_axis_name="c", subcore_axis_name="s")


def _hop_body(s_hbm, row3, col3, outa, outb,
              rv, colv, rows0, rows1, zbuf, acc, semz, sem0, sem1, semr):
    c = lax.axis_index("c")
    sid = lax.axis_index("s")
    wid = c * NS + sid

    for r in range(16):
        for k in range(D // 16):
            zbuf[r, pl.ds(k * 16, 16)] = jnp.zeros((16,), jnp.float32)

    zdescs = [pltpu.async_copy(zbuf, acc.at[pl.ds(sid * RPT + j * 16, 16)],
                               semz) for j in range(RPT // 16)]

    pltpu.sync_copy(col3.at[wid], colv)
    pltpu.sync_copy(row3.at[wid, pl.ds(0, 8)], rv.at[pl.ds(0, 8)])

    @pl.when(sid == NS - 1)
    def _():
        pltpu.async_copy(zbuf, acc.at[pl.ds(N - 16, 16)], semz).wait()
    for d in zdescs:
        d.wait()
    plsc.subcore_barrier()

    def _gather(slot, buf, sem):
        return pltpu.async_copy(s_hbm.at[rv.at[slot]], buf, sem)

    def _scatter(i, buf):
        pltpu.sync_copy(buf, acc.at[colv.at[i]], add=True)

    def _group(g, _):
        p = (g % 2) * 8
        gnext = pl.multiple_of(jnp.minimum(g + 1, NGRP - 1) * 8, 8)
        dpre = pltpu.async_copy(row3.at[wid, pl.ds(gnext, 8)],
                                rv.at[pl.ds(8 - p, 8)], semr)

        d0 = _gather(p, rows0, sem0)
        for k in range(4):
            i0 = g * 8 + 2 * k
            d1 = _gather(p + 2 * k + 1, rows1, sem1)
            d0.wait()
            _scatter(i0, rows0)
            if k < 3:
                d0 = _gather(p + 2 * k + 2, rows0, sem0)
            d1.wait()
            _scatter(i0 + 1, rows1)

        dpre.wait()
        return 0

    lax.fori_loop(0, NGRP, _group, 0)
    plsc.subcore_barrier()

    def _writeback(out):
        pltpu.sync_copy(acc.at[pl.ds(sid * RPT, RPT)],
                        out.at[pl.ds(sid * RPT, RPT)])

        @pl.when(sid == NS - 1)
        def _():
            pltpu.sync_copy(acc.at[pl.ds(N - 16, 16)],
                            out.at[pl.ds(N - 16, 16)])

    @pl.when(c == 0)
    def _():
        _writeback(outa)

    @pl.when(c == 1)
    def _():
        _writeback(outb)


_sc_hop_raw = functools.partial(
    pl.kernel,
    out_type=[jax.ShapeDtypeStruct((N, D), jnp.float32),
              jax.ShapeDtypeStruct((N, D), jnp.float32)],
    mesh=_mesh,
    scratch_types=[
        pltpu.VMEM((16, CH), jnp.int32),
        pltpu.VMEM((NCHUNK, CH), jnp.int32),
        pltpu.VMEM((CH, D), jnp.float32),
        pltpu.VMEM((CH, D), jnp.float32),
        pltpu.VMEM((16, D), jnp.float32),
        pltpu.VMEM_SHARED((N, D), jnp.float32),
        pltpu.SemaphoreType.DMA,
        pltpu.SemaphoreType.DMA,
        pltpu.SemaphoreType.DMA,
        pltpu.SemaphoreType.DMA,
    ],
)(_hop_body)


def _sc_hop(s, row3, col3):
    return _sc_hop_raw(s, row3, col3)


def _deg_body(col3, outa, outb, colv, ones, zbuf, acc, semz, sem0, sem1):
    c = lax.axis_index("c")
    sid = lax.axis_index("s")
    wid = c * NS + sid

    for r in range(16):
        for k in range(D // 16):
            zbuf[r, pl.ds(k * 16, 16)] = jnp.zeros((16,), jnp.float32)

    zdescs = [pltpu.async_copy(zbuf, acc.at[pl.ds(sid * RPT + j * 16, 16)],
                               semz) for j in range(RPT // 16)]
    pltpu.sync_copy(col3.at[wid], colv)

    def _fill(r, _):
        for k in range(D // 16):
            ones[r, pl.ds(k * 16, 16)] = jnp.ones((16,), jnp.float32)
        return 0

    lax.fori_loop(0, CH, _fill, 0)

    @pl.when(sid == NS - 1)
    def _():
        pltpu.async_copy(zbuf, acc.at[pl.ds(N - 16, 16)], semz).wait()
    for d in zdescs:
        d.wait()
    plsc.subcore_barrier()

    def _scat(i, sem):
        return pltpu.async_copy(ones, acc.at[colv.at[i]], sem, add=True)

    def _group(g, _):
        d0 = _scat(g * 8, sem0)
        for k in range(4):
            d1 = _scat(g * 8 + 2 * k + 1, sem1)
            d0.wait()
            if k < 3:
                d0 = _scat(g * 8 + 2 * k + 2, sem0)
            d1.wait()
        return 0

    lax.fori_loop(0, NGRP, _group, 0)
    plsc.subcore_barrier()

    def _writeback(out):
        pltpu.sync_copy(acc.at[pl.ds(sid * RPT, RPT)],
                        out.at[pl.ds(sid * RPT, RPT)])

        @pl.when(sid == NS - 1)
        def _():
            pltpu.sync_copy(acc.at[pl.ds(N - 16, 16)],
                            out.at[pl.ds(N - 16, 16)])

    @pl.when(c == 0)
    def _():
        _writeback(outa)

    @pl.when(c == 1)
    def _():
        _writeback(outb)


_sc_deg = functools.partial(
    pl.kernel,
    out_type=[jax.ShapeDtypeStruct((N, D), jnp.float32),
              jax.ShapeDtypeStruct((N, D), jnp.float32)],
    mesh=_mesh,
    scratch_types=[
        pltpu.VMEM((NCHUNK, CH), jnp.int32),
        pltpu.VMEM((CH, D), jnp.float32),
        pltpu.VMEM((16, D), jnp.float32),
        pltpu.VMEM_SHARED((N, D), jnp.float32),
        pltpu.SemaphoreType.DMA,
        pltpu.SemaphoreType.DMA,
        pltpu.SemaphoreType.DMA,
    ],
)(_deg_body)


def _scale_body(x_ref, d_ref, o_ref):
    o_ref[...] = x_ref[...] * d_ref[...]


_k_scale = pl.pallas_call(
    _scale_body,
    grid=(GRID,),
    in_specs=[pl.BlockSpec((BLK, D), lambda i: (i, 0)),
              pl.BlockSpec((BLK, 1), lambda i: (i, 0))],
    out_specs=pl.BlockSpec((BLK, D), lambda i: (i, 0)),
    out_shape=jax.ShapeDtypeStruct((N, D), jnp.float32),
)


def _mid_body(ta_ref, tb_ref, d2_ref, o_ref):
    o_ref[...] = d2_ref[...] * (ta_ref[...] + tb_ref[...])


_k_mid = pl.pallas_call(
    _mid_body,
    grid=(GRID,),
    in_specs=[pl.BlockSpec((BLK, D), lambda i: (i, 0)),
              pl.BlockSpec((BLK, D), lambda i: (i, 0)),
              pl.BlockSpec((BLK, 1), lambda i: (i, 0))],
    out_specs=pl.BlockSpec((BLK, D), lambda i: (i, 0)),
    out_shape=jax.ShapeDtypeStruct((N, D), jnp.float32),
)


def _make_step(cfac):
    def _step_body(hs_ref, hb_ref, t1a, t1b, t2a, t2b, d_ref,
                   w0, w1, w2, b_ref, ho_ref, so_ref):
        dv = d_ref[...]
        cur1 = dv * (t1a[...] + t1b[...])
        cur2 = dv * (t2a[...] + t2b[...])
        conv = jnp.dot(hs_ref[...], w0[...], preferred_element_type=jnp.float32)
        conv = conv + jnp.dot(cur1, w1[...], preferred_element_type=jnp.float32)
        conv = conv + jnp.dot(cur2, w2[...], preferred_element_type=jnp.float32)
        conv = conv + b_ref[...]
        ho = hb_ref[...] + cfac * jnp.tanh(conv)
        ho_ref[...] = ho
        so_ref[...] = dv * ho

    blk = pl.BlockSpec((BLK, D), lambda i: (i, 0))
    return pl.pallas_call(
        _step_body,
        grid=(GRID,),
        in_specs=[blk, blk, blk, blk, blk, blk,
                  pl.BlockSpec((BLK, 1), lambda i: (i, 0)),
                  pl.BlockSpec((D, D), lambda i: (0, 0)),
                  pl.BlockSpec((D, D), lambda i: (0, 0)),
                  pl.BlockSpec((D, D), lambda i: (0, 0)),
                  pl.BlockSpec((1, D), lambda i: (0, 0))],
        out_specs=[blk, blk],
        out_shape=[jax.ShapeDtypeStruct((N, D), jnp.float32),
                   jax.ShapeDtypeStruct((N, D), jnp.float32)],
    )


_k_step_mid = _make_step(0.5 * EPS)
_k_step_full = _make_step(EPS)


def _readout_body(hm_ref, wr_ref, br_ref, y_ref):
    y_ref[...] = (jnp.dot(hm_ref[...], wr_ref[...],
                          preferred_element_type=jnp.float32) + br_ref[...])


_k_readout = pl.pallas_call(
    _readout_body,
    grid=(GRID,),
    in_specs=[pl.BlockSpec((BLK, D), lambda i: (i, 0)),
              pl.BlockSpec((D, D), lambda i: (0, 0)),
              pl.BlockSpec((1, D), lambda i: (0, 0))],
    out_specs=pl.BlockSpec((BLK, D), lambda i: (i, 0)),
    out_shape=jax.ShapeDtypeStruct((N, D), jnp.float32),
)


def kernel(x, edge_index, delta_t, W0, W1, W2, b, Wr, br):
    row3 = edge_index[0].reshape(NW, NCHUNK, CH)
    col3 = edge_index[1].reshape(NW, NCHUNK, CH)

    dega, degb = _sc_deg(col3)
    deg = dega[:, 0] + degb[:, 0]
    dinv = jnp.where(deg > 0, lax.rsqrt(jnp.where(deg > 0, deg, 1.0)), 0.0)
    dcol = dinv.reshape(N, 1)
    d2col = dcol * dcol
    b2 = b.reshape(1, D)
    br2 = br.reshape(1, D)

    s0 = _k_scale(x, dcol)

    def _step(_, carry):
        h, hm, s = carry
        t1a, t1b = _sc_hop(s, row3, col3)
        s1 = _k_mid(t1a, t1b, d2col)
        t2a, t2b = _sc_hop(s1, row3, col3)
        hm_new, sm = _k_step_mid(h, h, t1a, t1b, t2a, t2b, dcol,
                                 W0, W1, W2, b2)
        t3a, t3b = _sc_hop(sm, row3, col3)
        s3 = _k_mid(t3a, t3b, d2col)
        t4a, t4b = _sc_hop(s3, row3, col3)
        h_new, s_new = _k_step_full(hm_new, h, t3a, t3b, t4a, t4b, dcol,
                                    W0, W1, W2, b2)
        return (h_new, hm_new, s_new)

    h, hm, _ = lax.fori_loop(0, delta_t, _step, (x, x, s0))
    y = _k_readout(hm, Wr, br2)
    return (y, hm)


# speedup vs baseline: 1.0421x; 1.0195x over previous
import functools

import jax
import jax.numpy as jnp
from jax import lax
from jax.experimental import pallas as pl
from jax.experimental.pallas import tpu as pltpu
from jax.experimental.pallas import tpu_sc as plsc

EPS = 0.1
N = 10000
D = 128
E = 320000
NC = 2
NS = 16
NW = NC * NS
EPT = E // NW
CH = 125
NCHUNK = EPT // CH
NGRP = NCHUNK // 8
RPT = 624
BLK = 2000
GRID = N // BLK

_mesh = plsc.VectorSubcoreMesh(core_axis_name="c", subcore_axis_name="s")


def _hop_body(s_hbm, row3, col3, outa, outb,
              rv, colv, rows0, rows1, zbuf, acc, semz, sem0, sem1, semr):
    c = lax.axis_index("c")
    sid = lax.axis_index("s")
    wid = c * NS + sid

    for r in range(16):
        for k in range(D // 16):
            zbuf[r, pl.ds(k * 16, 16)] = jnp.zeros((16,), jnp.float32)

    zdescs = [pltpu.async_copy(zbuf, acc.at[pl.ds(sid * RPT + j * 16, 16)],
                               semz) for j in range(RPT // 16)]

    pltpu.sync_copy(col3.at[wid], colv)
    pltpu.sync_copy(row3.at[wid, pl.ds(0, 8)], rv.at[pl.ds(0, 8)])

    @pl.when(sid == NS - 1)
    def _():
        pltpu.async_copy(zbuf, acc.at[pl.ds(N - 16, 16)], semz).wait()
    for d in zdescs:
        d.wait()
    plsc.subcore_barrier()

    def _gather(slot, buf, sem):
        return pltpu.async_copy(s_hbm.at[rv.at[slot]], buf, sem)

    def _scatter(i, buf):
        pltpu.sync_copy(buf, acc.at[colv.at[i]], add=True)

    def _group(g, _):
        p = (g % 2) * 8
        gnext = pl.multiple_of(jnp.minimum(g + 1, NGRP - 1) * 8, 8)
        dpre = pltpu.async_copy(row3.at[wid, pl.ds(gnext, 8)],
                                rv.at[pl.ds(8 - p, 8)], semr)

        d0 = _gather(p, rows0, sem0)
        for k in range(4):
            i0 = g * 8 + 2 * k
            d1 = _gather(p + 2 * k + 1, rows1, sem1)
            d0.wait()
            _scatter(i0, rows0)
            if k < 3:
                d0 = _gather(p + 2 * k + 2, rows0, sem0)
            d1.wait()
            _scatter(i0 + 1, rows1)

        dpre.wait()
        return 0

    lax.fori_loop(0, NGRP, _group, 0)
    plsc.subcore_barrier()

    def _writeback(out):
        pltpu.sync_copy(acc.at[pl.ds(sid * RPT, RPT)],
                        out.at[pl.ds(sid * RPT, RPT)])

        @pl.when(sid == NS - 1)
        def _():
            pltpu.sync_copy(acc.at[pl.ds(N - 16, 16)],
                            out.at[pl.ds(N - 16, 16)])

    @pl.when(c == 0)
    def _():
        _writeback(outa)

    @pl.when(c == 1)
    def _():
        _writeback(outb)


_sc_hop_raw = functools.partial(
    pl.kernel,
    out_type=[jax.ShapeDtypeStruct((N, D), jnp.float32),
              jax.ShapeDtypeStruct((N, D), jnp.float32)],
    mesh=_mesh,
    scratch_types=[
        pltpu.VMEM((16, CH), jnp.int32),
        pltpu.VMEM((NCHUNK, CH), jnp.int32),
        pltpu.VMEM((CH, D), jnp.float32),
        pltpu.VMEM((CH, D), jnp.float32),
        pltpu.VMEM((16, D), jnp.float32),
        pltpu.VMEM_SHARED((N, D), jnp.float32),
        pltpu.SemaphoreType.DMA,
        pltpu.SemaphoreType.DMA,
        pltpu.SemaphoreType.DMA,
        pltpu.SemaphoreType.DMA,
    ],
)(_hop_body)


def _sc_hop(s, row3, col3):
    return _sc_hop_raw(s, row3, col3)


def _deg_body(col3, outa, outb, colv, ones, zbuf, acc, semz, sem0, sem1):
    c = lax.axis_index("c")
    sid = lax.axis_index("s")
    wid = c * NS + sid

    for r in range(16):
        for k in range(D // 16):
            zbuf[r, pl.ds(k * 16, 16)] = jnp.zeros((16,), jnp.float32)

    zdescs = [pltpu.async_copy(zbuf, acc.at[pl.ds(sid * RPT + j * 16, 16)],
                               semz) for j in range(RPT // 16)]
    pltpu.sync_copy(col3.at[wid], colv)

    def _fill(r, _):
        for k in range(D // 16):
            ones[r, pl.ds(k * 16, 16)] = jnp.ones((16,), jnp.float32)
        return 0

    lax.fori_loop(0, CH, _fill, 0)

    @pl.when(sid == NS - 1)
    def _():
        pltpu.async_copy(zbuf, acc.at[pl.ds(N - 16, 16)], semz).wait()
    for d in zdescs:
        d.wait()
    plsc.subcore_barrier()

    def _scat(i, sem):
        return pltpu.async_copy(ones, acc.at[colv.at[i]], sem, add=True)

    def _group(g, _):
        d0 = _scat(g * 8, sem0)
        for k in range(4):
            d1 = _scat(g * 8 + 2 * k + 1, sem1)
            d0.wait()
            if k < 3:
                d0 = _scat(g * 8 + 2 * k + 2, sem0)
            d1.wait()
        return 0

    lax.fori_loop(0, NGRP, _group, 0)
    plsc.subcore_barrier()

    def _writeback(out):
        pltpu.sync_copy(acc.at[pl.ds(sid * RPT, RPT)],
                        out.at[pl.ds(sid * RPT, RPT)])

        @pl.when(sid == NS - 1)
        def _():
            pltpu.sync_copy(acc.at[pl.ds(N - 16, 16)],
                            out.at[pl.ds(N - 16, 16)])

    @pl.when(c == 0)
    def _():
        _writeback(outa)

    @pl.when(c == 1)
    def _():
        _writeback(outb)


_sc_deg = functools.partial(
    pl.kernel,
    out_type=[jax.ShapeDtypeStruct((N, D), jnp.float32),
              jax.ShapeDtypeStruct((N, D), jnp.float32)],
    mesh=_mesh,
    scratch_types=[
        pltpu.VMEM((NCHUNK, CH), jnp.int32),
        pltpu.VMEM((CH, D), jnp.float32),
        pltpu.VMEM((16, D), jnp.float32),
        pltpu.VMEM_SHARED((N, D), jnp.float32),
        pltpu.SemaphoreType.DMA,
        pltpu.SemaphoreType.DMA,
        pltpu.SemaphoreType.DMA,
    ],
)(_deg_body)


def _scale_body(x_ref, d_ref, o_ref):
    o_ref[...] = x_ref[...] * d_ref[...]


_k_scale = pl.pallas_call(
    _scale_body,
    grid=(GRID,),
    in_specs=[pl.BlockSpec((BLK, D), lambda i: (i, 0)),
              pl.BlockSpec((BLK, 1), lambda i: (i, 0))],
    out_specs=pl.BlockSpec((BLK, D), lambda i: (i, 0)),
    out_shape=jax.ShapeDtypeStruct((N, D), jnp.float32),
)


def _mid_body(ta_ref, tb_ref, d2_ref, o_ref):
    o_ref[...] = d2_ref[...] * (ta_ref[...] + tb_ref[...])


_k_mid = pl.pallas_call(
    _mid_body,
    grid=(GRID,),
    in_specs=[pl.BlockSpec((BLK, D), lambda i: (i, 0)),
              pl.BlockSpec((BLK, D), lambda i: (i, 0)),
              pl.BlockSpec((BLK, 1), lambda i: (i, 0))],
    out_specs=pl.BlockSpec((BLK, D), lambda i: (i, 0)),
    out_shape=jax.ShapeDtypeStruct((N, D), jnp.float32),
)


def _make_step(cfac):
    def _step_body(hs_ref, hb_ref, t1a, t1b, t2a, t2b, d_ref,
                   w0, w1, w2, b_ref, ho_ref, so_ref):
        dv = d_ref[...]
        cur1 = dv * (t1a[...] + t1b[...])
        cur2 = dv * (t2a[...] + t2b[...])
        conv = jnp.dot(hs_ref[...], w0[...], preferred_element_type=jnp.float32)
        conv = conv + jnp.dot(cur1, w1[...], preferred_element_type=jnp.float32)
        conv = conv + jnp.dot(cur2, w2[...], preferred_element_type=jnp.float32)
        conv = conv + b_ref[...]
        ho = hb_ref[...] + cfac * jnp.tanh(conv)
        ho_ref[...] = ho
        so_ref[...] = dv * ho

    blk = pl.BlockSpec((BLK, D), lambda i: (i, 0))
    return pl.pallas_call(
        _step_body,
        grid=(GRID,),
        in_specs=[blk, blk, blk, blk, blk, blk,
                  pl.BlockSpec((BLK, 1), lambda i: (i, 0)),
                  pl.BlockSpec((D, D), lambda i: (0, 0)),
                  pl.BlockSpec((D, D), lambda i: (0, 0)),
                  pl.BlockSpec((D, D), lambda i: (0, 0)),
                  pl.BlockSpec((1, D), lambda i: (0, 0))],
        out_specs=[blk, blk],
        out_shape=[jax.ShapeDtypeStruct((N, D), jnp.float32),
                   jax.ShapeDtypeStruct((N, D), jnp.float32)],
    )


_k_step_mid = _make_step(0.5 * EPS)
_k_step_full = _make_step(EPS)


def _readout_body(hm_ref, wr_ref, br_ref, y_ref):
    y_ref[...] = (jnp.dot(hm_ref[...], wr_ref[...],
                          preferred_element_type=jnp.float32) + br_ref[...])


_k_readout = pl.pallas_call(
    _readout_body,
    grid=(GRID,),
    in_specs=[pl.BlockSpec((BLK, D), lambda i: (i, 0)),
              pl.BlockSpec((D, D), lambda i: (0, 0)),
              pl.BlockSpec((1, D), lambda i: (0, 0))],
    out_specs=pl.BlockSpec((BLK, D), lambda i: (i, 0)),
    out_shape=jax.ShapeDtypeStruct((N, D), jnp.float32),
)


def kernel(x, edge_index, delta_t, W0, W1, W2, b, Wr, br):
    row3 = edge_index[0].reshape(NW, NCHUNK, CH)
    col3 = edge_index[1].reshape(NW, NCHUNK, CH)

    dega, degb = _sc_deg(col3)
    deg = dega[:, 0] + degb[:, 0]
    dinv = jnp.where(deg > 0, lax.rsqrt(jnp.where(deg > 0, deg, 1.0)), 0.0)
    dcol = dinv.reshape(N, 1)
    d2col = dcol * dcol
    b2 = b.reshape(1, D)
    br2 = br.reshape(1, D)

    s0 = _k_scale(x, dcol)

    def _step(_, carry):
        h, hm, s = carry
        t1a, t1b = _sc_hop(s, row3, col3)
        s1 = _k_mid(t1a, t1b, d2col)
        t2a, t2b = _sc_hop(s1, row3, col3)
        hm_new, sm = _k_step_mid(h, h, t1a, t1b, t2a, t2b, dcol,
                                 W0, W1, W2, b2)
        t3a, t3b = _sc_hop(sm, row3, col3)
        s3 = _k_mid(t3a, t3b, d2col)
        t4a, t4b = _sc_hop(s3, row3, col3)
        h_new, s_new = _k_step_full(hm_new, h, t3a, t3b, t4a, t4b, dcol,
                                    W0, W1, W2, b2)
        return (h_new, hm_new, s_new)

    h, hm, _ = lax.fori_loop(0, delta_t, _step, (x, x, s0))
    y = _k_readout(hm, Wr, br2)
    return (y, hm)


# speedup vs baseline: 1.0431x; 1.0009x over previous
import functools

import jax
import jax.numpy as jnp
from jax import lax
from jax.experimental import pallas as pl
from jax.experimental.pallas import tpu as pltpu
from jax.experimental.pallas import tpu_sc as plsc

EPS = 0.1
N = 10000
D = 128
E = 320000
NC = 2
NS = 16
NW = NC * NS
EPT = E // NW
CH = 125
NCHUNK = EPT // CH
NGRP = NCHUNK // 8
RPT = 624
BLK = 5000
GRID = N // BLK

_mesh = plsc.VectorSubcoreMesh(core_axis_name="c", subcore_axis_name="s")


def _hop_body(s_hbm, row3, col3, outa, outb,
              rv, colv, rows0, rows1, zbuf, acc, semz, sem0, sem1, semr):
    c = lax.axis_index("c")
    sid = lax.axis_index("s")
    wid = c * NS + sid

    for r in range(16):
        for k in range(D // 16):
            zbuf[r, pl.ds(k * 16, 16)] = jnp.zeros((16,), jnp.float32)

    zdescs = [pltpu.async_copy(zbuf, acc.at[pl.ds(sid * RPT + j * 16, 16)],
                               semz) for j in range(RPT // 16)]

    pltpu.sync_copy(col3.at[wid], colv)
    pltpu.sync_copy(row3.at[wid, pl.ds(0, 8)], rv.at[pl.ds(0, 8)])

    @pl.when(sid == NS - 1)
    def _():
        pltpu.async_copy(zbuf, acc.at[pl.ds(N - 16, 16)], semz).wait()
    for d in zdescs:
        d.wait()
    plsc.subcore_barrier()

    def _gather(slot, buf, sem):
        return pltpu.async_copy(s_hbm.at[rv.at[slot]], buf, sem)

    def _scatter(i, buf):
        pltpu.sync_copy(buf, acc.at[colv.at[i]], add=True)

    def _group(g, _):
        p = (g % 2) * 8
        gnext = pl.multiple_of(jnp.minimum(g + 1, NGRP - 1) * 8, 8)
        dpre = pltpu.async_copy(row3.at[wid, pl.ds(gnext, 8)],
                                rv.at[pl.ds(8 - p, 8)], semr)

        d0 = _gather(p, rows0, sem0)
        for k in range(4):
            i0 = g * 8 + 2 * k
            d1 = _gather(p + 2 * k + 1, rows1, sem1)
            d0.wait()
            _scatter(i0, rows0)
            if k < 3:
                d0 = _gather(p + 2 * k + 2, rows0, sem0)
            d1.wait()
            _scatter(i0 + 1, rows1)

        dpre.wait()
        return 0

    lax.fori_loop(0, NGRP, _group, 0)
    plsc.subcore_barrier()

    def _writeback(out):
        pltpu.sync_copy(acc.at[pl.ds(sid * RPT, RPT)],
                        out.at[pl.ds(sid * RPT, RPT)])

        @pl.when(sid == NS - 1)
        def _():
            pltpu.sync_copy(acc.at[pl.ds(N - 16, 16)],
                            out.at[pl.ds(N - 16, 16)])

    @pl.when(c == 0)
    def _():
        _writeback(outa)

    @pl.when(c == 1)
    def _():
        _writeback(outb)


_sc_hop_raw = functools.partial(
    pl.kernel,
    out_type=[jax.ShapeDtypeStruct((N, D), jnp.float32),
              jax.ShapeDtypeStruct((N, D), jnp.float32)],
    mesh=_mesh,
    scratch_types=[
        pltpu.VMEM((16, CH), jnp.int32),
        pltpu.VMEM((NCHUNK, CH), jnp.int32),
        pltpu.VMEM((CH, D), jnp.float32),
        pltpu.VMEM((CH, D), jnp.float32),
        pltpu.VMEM((16, D), jnp.float32),
        pltpu.VMEM_SHARED((N, D), jnp.float32),
        pltpu.SemaphoreType.DMA,
        pltpu.SemaphoreType.DMA,
        pltpu.SemaphoreType.DMA,
        pltpu.SemaphoreType.DMA,
    ],
)(_hop_body)


def _sc_hop(s, row3, col3):
    return _sc_hop_raw(s, row3, col3)


def _deg_body(col3, outa, outb, colv, ones, zbuf, acc, semz, sem0, sem1):
    c = lax.axis_index("c")
    sid = lax.axis_index("s")
    wid = c * NS + sid

    for r in range(16):
        for k in range(D // 16):
            zbuf[r, pl.ds(k * 16, 16)] = jnp.zeros((16,), jnp.float32)

    zdescs = [pltpu.async_copy(zbuf, acc.at[pl.ds(sid * RPT + j * 16, 16)],
                               semz) for j in range(RPT // 16)]
    pltpu.sync_copy(col3.at[wid], colv)

    def _fill(r, _):
        for k in range(D // 16):
            ones[r, pl.ds(k * 16, 16)] = jnp.ones((16,), jnp.float32)
        return 0

    lax.fori_loop(0, CH, _fill, 0)

    @pl.when(sid == NS - 1)
    def _():
        pltpu.async_copy(zbuf, acc.at[pl.ds(N - 16, 16)], semz).wait()
    for d in zdescs:
        d.wait()
    plsc.subcore_barrier()

    def _scat(i, sem):
        return pltpu.async_copy(ones, acc.at[colv.at[i]], sem, add=True)

    def _group(g, _):
        d0 = _scat(g * 8, sem0)
        for k in range(4):
            d1 = _scat(g * 8 + 2 * k + 1, sem1)
            d0.wait()
            if k < 3:
                d0 = _scat(g * 8 + 2 * k + 2, sem0)
            d1.wait()
        return 0

    lax.fori_loop(0, NGRP, _group, 0)
    plsc.subcore_barrier()

    def _writeback(out):
        pltpu.sync_copy(acc.at[pl.ds(sid * RPT, RPT)],
                        out.at[pl.ds(sid * RPT, RPT)])

        @pl.when(sid == NS - 1)
        def _():
            pltpu.sync_copy(acc.at[pl.ds(N - 16, 16)],
                            out.at[pl.ds(N - 16, 16)])

    @pl.when(c == 0)
    def _():
        _writeback(outa)

    @pl.when(c == 1)
    def _():
        _writeback(outb)


_sc_deg = functools.partial(
    pl.kernel,
    out_type=[jax.ShapeDtypeStruct((N, D), jnp.float32),
              jax.ShapeDtypeStruct((N, D), jnp.float32)],
    mesh=_mesh,
    scratch_types=[
        pltpu.VMEM((NCHUNK, CH), jnp.int32),
        pltpu.VMEM((CH, D), jnp.float32),
        pltpu.VMEM((16, D), jnp.float32),
        pltpu.VMEM_SHARED((N, D), jnp.float32),
        pltpu.SemaphoreType.DMA,
        pltpu.SemaphoreType.DMA,
        pltpu.SemaphoreType.DMA,
    ],
)(_deg_body)


def _scale_body(x_ref, d_ref, o_ref):
    o_ref[...] = x_ref[...] * d_ref[...]


_k_scale = pl.pallas_call(
    _scale_body,
    grid=(GRID,),
    in_specs=[pl.BlockSpec((BLK, D), lambda i: (i, 0)),
              pl.BlockSpec((BLK, 1), lambda i: (i, 0))],
    out_specs=pl.BlockSpec((BLK, D), lambda i: (i, 0)),
    out_shape=jax.ShapeDtypeStruct((N, D), jnp.float32),
)


def _mid_body(ta_ref, tb_ref, d2_ref, o_ref):
    o_ref[...] = d2_ref[...] * (ta_ref[...] + tb_ref[...])


_k_mid = pl.pallas_call(
    _mid_body,
    grid=(GRID,),
    in_specs=[pl.BlockSpec((BLK, D), lambda i: (i, 0)),
              pl.BlockSpec((BLK, D), lambda i: (i, 0)),
              pl.BlockSpec((BLK, 1), lambda i: (i, 0))],
    out_specs=pl.BlockSpec((BLK, D), lambda i: (i, 0)),
    out_shape=jax.ShapeDtypeStruct((N, D), jnp.float32),
)


def _make_step(cfac):
    def _step_body(hs_ref, hb_ref, t1a, t1b, t2a, t2b, d_ref,
                   w0, w1, w2, b_ref, ho_ref, so_ref):
        dv = d_ref[...]
        cur1 = dv * (t1a[...] + t1b[...])
        cur2 = dv * (t2a[...] + t2b[...])
        conv = jnp.dot(hs_ref[...], w0[...], preferred_element_type=jnp.float32)
        conv = conv + jnp.dot(cur1, w1[...], preferred_element_type=jnp.float32)
        conv = conv + jnp.dot(cur2, w2[...], preferred_element_type=jnp.float32)
        conv = conv + b_ref[...]
        ho = hb_ref[...] + cfac * jnp.tanh(conv)
        ho_ref[...] = ho
        so_ref[...] = dv * ho

    blk = pl.BlockSpec((BLK, D), lambda i: (i, 0))
    return pl.pallas_call(
        _step_body,
        grid=(GRID,),
        in_specs=[blk, blk, blk, blk, blk, blk,
                  pl.BlockSpec((BLK, 1), lambda i: (i, 0)),
                  pl.BlockSpec((D, D), lambda i: (0, 0)),
                  pl.BlockSpec((D, D), lambda i: (0, 0)),
                  pl.BlockSpec((D, D), lambda i: (0, 0)),
                  pl.BlockSpec((1, D), lambda i: (0, 0))],
        out_specs=[blk, blk],
        out_shape=[jax.ShapeDtypeStruct((N, D), jnp.float32),
                   jax.ShapeDtypeStruct((N, D), jnp.float32)],
    )


_k_step_mid = _make_step(0.5 * EPS)
_k_step_full = _make_step(EPS)


def _readout_body(hm_ref, wr_ref, br_ref, y_ref):
    y_ref[...] = (jnp.dot(hm_ref[...], wr_ref[...],
                          preferred_element_type=jnp.float32) + br_ref[...])


_k_readout = pl.pallas_call(
    _readout_body,
    grid=(GRID,),
    in_specs=[pl.BlockSpec((BLK, D), lambda i: (i, 0)),
              pl.BlockSpec((D, D), lambda i: (0, 0)),
              pl.BlockSpec((1, D), lambda i: (0, 0))],
    out_specs=pl.BlockSpec((BLK, D), lambda i: (i, 0)),
    out_shape=jax.ShapeDtypeStruct((N, D), jnp.float32),
)


def kernel(x, edge_index, delta_t, W0, W1, W2, b, Wr, br):
    row3 = edge_index[0].reshape(NW, NCHUNK, CH)
    col3 = edge_index[1].reshape(NW, NCHUNK, CH)

    dega, degb = _sc_deg(col3)
    deg = dega[:, 0] + degb[:, 0]
    dinv = jnp.where(deg > 0, lax.rsqrt(jnp.where(deg > 0, deg, 1.0)), 0.0)
    dcol = dinv.reshape(N, 1)
    d2col = dcol * dcol
    b2 = b.reshape(1, D)
    br2 = br.reshape(1, D)

    s0 = _k_scale(x, dcol)

    def _step(_, carry):
        h, hm, s = carry
        t1a, t1b = _sc_hop(s, row3, col3)
        s1 = _k_mid(t1a, t1b, d2col)
        t2a, t2b = _sc_hop(s1, row3, col3)
        hm_new, sm = _k_step_mid(h, h, t1a, t1b, t2a, t2b, dcol,
                                 W0, W1, W2, b2)
        t3a, t3b = _sc_hop(sm, row3, col3)
        s3 = _k_mid(t3a, t3b, d2col)
        t4a, t4b = _sc_hop(s3, row3, col3)
        h_new, s_new = _k_step_full(hm_new, h, t3a, t3b, t4a, t4b, dcol,
                                    W0, W1, W2, b2)
        return (h_new, hm_new, s_new)

    h, hm, _ = lax.fori_loop(0, delta_t, _step, (x, x, s0))
    y = _k_readout(hm, Wr, br2)
    return (y, hm)
